# Initial kernel scaffold; baseline (speedup 1.0000x reference)
import jax
import jax.numpy as jnp
from jax.experimental import pallas as pl


def kernel(inputs):
    raise NotImplementedError("write your pallas kernel here")


# speedup vs baseline: 4.6506x; 4.6506x over previous
import numpy as np
import jax
import jax.numpy as jnp
from jax.experimental import pallas as pl

_N_ROWS = 16384
_N_COLS = 330
_BLOCK_ROWS = 1024


def _fix_mask() -> np.ndarray:
    fix = np.zeros((_N_COLS,), dtype=bool)
    for h in range(12):
        base = 26 * h
        for k in range(12):
            fix[base + 2 * k + 1] = True
    for k in range(4):
        fix[312 + 2 * k + 1] = True
        fix[320 + 2 * k + 1] = True
    return fix


_FIX = _fix_mask()


def _correct_block(x_ref, mask_ref, out_ref):
    x = x_ref[...]
    prev = jnp.roll(x, 1, axis=1)
    fix = mask_ref[...] != 0.0
    out_ref[...] = jnp.where(fix & (prev == 0.0), 0.0, x)


def kernel(inputs):
    n_rows, n_cols = inputs.shape
    mask = jnp.asarray(
        np.broadcast_to(_FIX.astype(np.float32), (_BLOCK_ROWS, _N_COLS))
    )
    grid = (n_rows // _BLOCK_ROWS,)
    return pl.pallas_call(
        _correct_block,
        grid=grid,
        in_specs=[
            pl.BlockSpec((_BLOCK_ROWS, n_cols), lambda i: (i, 0)),
            pl.BlockSpec((_BLOCK_ROWS, n_cols), lambda i: (0, 0)),
        ],
        out_specs=pl.BlockSpec((_BLOCK_ROWS, n_cols), lambda i: (i, 0)),
        out_shape=jax.ShapeDtypeStruct((n_rows, n_cols), inputs.dtype),
    )(inputs, mask)


# speedup vs baseline: 4.7764x; 1.0270x over previous
import numpy as np
import jax
import jax.numpy as jnp
from jax.experimental import pallas as pl

_N_ROWS = 16384
_N_COLS = 330
_BLOCK_ROWS = 1024


def _fix_mask() -> np.ndarray:
    fix = np.zeros((_N_COLS,), dtype=bool)
    for h in range(12):
        base = 26 * h
        for k in range(12):
            fix[base + 2 * k + 1] = True
    for k in range(4):
        fix[312 + 2 * k + 1] = True
        fix[320 + 2 * k + 1] = True
    return fix


_FIX = _fix_mask()


def _correct_block(x_ref, out_ref):
    x = x_ref[...]
    prev = jnp.roll(x, 1, axis=1)
    col = jax.lax.broadcasted_iota(jnp.int32, x.shape, 1)
    is_odd = (col & 1) == 1
    house = (col < 312) & ((col % 26) < 24)
    mid = (col >= 312) & (col < 328)
    fix = is_odd & (house | mid)
    out_ref[...] = jnp.where(fix & (prev == 0.0), 0.0, x)


def kernel(inputs):
    n_rows, n_cols = inputs.shape
    grid = (n_rows // _BLOCK_ROWS,)
    return pl.pallas_call(
        _correct_block,
        grid=grid,
        in_specs=[
            pl.BlockSpec((_BLOCK_ROWS, n_cols), lambda i: (i, 0)),
        ],
        out_specs=pl.BlockSpec((_BLOCK_ROWS, n_cols), lambda i: (i, 0)),
        out_shape=jax.ShapeDtypeStruct((n_rows, n_cols), inputs.dtype),
    )(inputs)
